# baseline (speedup 1.0000x reference)
import functools

import jax
import jax.numpy as jnp
from jax import lax
from jax.experimental import pallas as pl
from jax.experimental.pallas import tpu as pltpu
from jax.experimental.pallas import tpu_sc as plsc

_D = 64
_V = 1_000_000
_B = 16384
_NW = 32
_BPW = _B // _NW
_CHUNK = 128
_NCHUNK = _BPW * _D // _CHUNK
_FIRE = 8


def _body(wflat_hbm, idx_hbm, out_hbm, idx_v, addr_v, rows_v, sem):
    wid = lax.axis_index("s") * 2 + lax.axis_index("c")
    base_b = wid * _BPW

    pltpu.sync_copy(idx_hbm.at[pl.ds(base_b, _BPW)], idx_v)

    lane = lax.iota(jnp.int32, 16)
    offs = [(lane + 16 * j) * _V for j in range(4)]

    def gen(g, _):
        iv = idx_v[pl.ds(g * 16, 16)]
        for l in range(16):
            bvec = jnp.full((16,), iv[l], jnp.int32)
            b = g * 16 + l
            row = b >> 1
            colb = (l & 1) * 64
            for j in range(4):
                addr_v[row, pl.ds(colb + 16 * j, 16)] = bvec + offs[j]
        return 0

    lax.fori_loop(0, _BPW // 16, gen, 0)

    def fire(g, _):
        c0 = g * _FIRE
        for t in range(_FIRE):
            pltpu.make_async_copy(
                wflat_hbm.at[addr_v.at[c0 + t]], rows_v.at[c0 + t], sem
            ).start()
        for t in range(_FIRE):
            pltpu.make_async_copy(
                wflat_hbm.at[addr_v.at[c0 + t]], rows_v.at[c0 + t], sem
            ).wait()
        return 0

    lax.fori_loop(0, _NCHUNK // _FIRE, fire, 0)

    pltpu.sync_copy(rows_v, out_hbm.at[pl.ds(wid * _NCHUNK, _NCHUNK)])


@jax.jit
def kernel(index, weights):
    wflat = weights.reshape(_D * _V)
    idx32 = index.astype(jnp.int32)
    run = pl.kernel(
        _body,
        out_type=jax.ShapeDtypeStruct((_B * _D // _CHUNK, _CHUNK), jnp.float32),
        mesh=plsc.VectorSubcoreMesh(core_axis_name="c", subcore_axis_name="s"),
        scratch_types=[
            pltpu.VMEM((_BPW,), jnp.int32),
            pltpu.VMEM((_NCHUNK, _CHUNK), jnp.int32),
            pltpu.VMEM((_NCHUNK, _CHUNK), jnp.float32),
            pltpu.SemaphoreType.DMA,
        ],
    )
    out = run(wflat, idx32)
    return out.reshape(_B, _D)


# speedup vs baseline: 1.0069x; 1.0069x over previous
import jax
import jax.numpy as jnp
from jax import lax
from jax.experimental import pallas as pl
from jax.experimental.pallas import tpu as pltpu
from jax.experimental.pallas import tpu_sc as plsc

_D = 64
_V = 1_000_000
_B = 16384
_NW = 32
_BPW = _B // _NW
_EPW = _BPW * _D


def _body(wflat_hbm, idx_hbm, out_hbm, idx_v, addr_v, rows_v, sem):
    wid = lax.axis_index("s") * 2 + lax.axis_index("c")
    base_b = wid * _BPW

    pltpu.sync_copy(idx_hbm.at[pl.ds(base_b, _BPW)], idx_v)

    lane = lax.iota(jnp.int32, 16)
    offs = [(lane + 16 * j) * _V for j in range(4)]

    def gen(g, _):
        iv = idx_v[pl.ds(g * 16, 16)]
        for l in range(16):
            bvec = jnp.full((16,), iv[l], jnp.int32)
            for j in range(4):
                addr_v[pl.ds(g * 1024 + l * 64 + 16 * j, 16)] = bvec + offs[j]
        return 0

    lax.fori_loop(0, _BPW // 16, gen, 0)

    pltpu.make_async_copy(wflat_hbm.at[addr_v], rows_v, sem).start()
    pltpu.make_async_copy(wflat_hbm.at[addr_v], rows_v, sem).wait()

    pltpu.sync_copy(rows_v, out_hbm.at[pl.ds(wid * _EPW, _EPW)])


@jax.jit
def kernel(index, weights):
    wflat = weights.reshape(_D * _V)
    idx32 = index.astype(jnp.int32)
    run = pl.kernel(
        _body,
        out_type=jax.ShapeDtypeStruct((_B * _D,), jnp.float32),
        mesh=plsc.VectorSubcoreMesh(core_axis_name="c", subcore_axis_name="s"),
        scratch_types=[
            pltpu.VMEM((_BPW,), jnp.int32),
            pltpu.VMEM((_EPW,), jnp.int32),
            pltpu.VMEM((_EPW,), jnp.float32),
            pltpu.SemaphoreType.DMA,
        ],
    )
    out = run(wflat, idx32)
    return out.reshape(_B, _D)
